# baseline (speedup 1.0000x reference)
import functools

import jax
import jax.numpy as jnp
from jax import lax
from jax.experimental import pallas as pl
from jax.experimental.pallas import tpu as pltpu
from jax.experimental.pallas import tpu_sc as plsc

N_POINTS = 1048576
N_FACES = 1538

NUM_CORES = 2
NUM_SUBCORES = 16
NW = NUM_CORES * NUM_SUBCORES
PTS_PER_W = N_POINTS // NW
CHUNK = 2048
N_CHUNKS = PTS_PER_W // CHUNK
GROUPS = CHUNK // 16
FPAD = 1544

_mesh = plsc.VectorSubcoreMesh(
    core_axis_name="c", subcore_axis_name="s", num_cores=NUM_CORES
)


@functools.partial(
    pl.kernel,
    out_type=jax.ShapeDtypeStruct((2 * N_POINTS,), jnp.float32),
    mesh=_mesh,
    compiler_params=pltpu.CompilerParams(needs_layout_passes=False),
    scratch_types=[
        [pltpu.VMEM((FPAD,), jnp.float32) for _ in range(6)],
        [pltpu.VMEM((CHUNK,), jnp.int32) for _ in range(2)],
        [pltpu.VMEM((3 * CHUNK,), jnp.float32) for _ in range(2)],
        [pltpu.VMEM((2 * CHUNK,), jnp.float32) for _ in range(2)],
        [pltpu.SemaphoreType.DMA for _ in range(2)],
        [pltpu.SemaphoreType.DMA for _ in range(2)],
        pltpu.SemaphoreType.DMA,
    ],
)
def _uv_kernel(
    table_hbm, fid_hbm, bary_hbm, out_hbm, table_v, fid_v, bary_v, uv_v,
    sem_in, sem_out, sem_tab,
):
    wid = lax.axis_index("s") * NUM_CORES + lax.axis_index("c")
    base0 = wid * PTS_PER_W
    t0_v, t1_v, t2_v, t3_v, t4_v, t5_v = table_v

    def start_in(ci, bi):
        base = base0 + ci * CHUNK
        return [
            pltpu.async_copy(fid_hbm.at[pl.ds(base, CHUNK)], fid_v[bi], sem_in[bi]),
            pltpu.async_copy(
                bary_hbm.at[pl.ds(3 * base, 3 * CHUNK)], bary_v[bi], sem_in[bi]
            ),
        ]

    pending_tab = [
        pltpu.async_copy(table_hbm.at[pl.ds(p * FPAD, FPAD)], table_v[p], sem_tab)
        for p in range(6)
    ]
    pending_in = {0: start_in(0, 0)}
    pending_out = {}
    for ci in range(N_CHUNKS):
        bi = ci % 2
        if ci + 1 < N_CHUNKS:
            pending_in[ci + 1] = start_in(ci + 1, 1 - bi)
        for d in pending_tab:
            d.wait()
        pending_tab = []
        for d in pending_in.pop(ci):
            d.wait()
        if ci - 2 in pending_out:
            pending_out.pop(ci - 2).wait()

        bary_b, fid_b, uv_b = bary_v[bi], fid_v[bi], uv_v[bi]

        @plsc.parallel_loop(0, GROUPS, unroll=8)
        def group_body(g):
            fid = fid_b[pl.ds(g * 16, 16)]
            t0 = plsc.load_gather(t0_v, [fid])
            t1 = plsc.load_gather(t1_v, [fid])
            t2 = plsc.load_gather(t2_v, [fid])
            t3 = plsc.load_gather(t3_v, [fid])
            t4 = plsc.load_gather(t4_v, [fid])
            t5 = plsc.load_gather(t5_v, [fid])
            blk, r = g // 8, (g % 8) * 16
            off_b = blk * 384 + r
            b0 = bary_b[pl.ds(off_b, 16)]
            b1 = bary_b[pl.ds(off_b + 128, 16)]
            b2 = bary_b[pl.ds(off_b + 256, 16)]
            off_u = blk * 256 + r
            uv_b[pl.ds(off_u, 16)] = t0 * b0 + t2 * b1 + t4 * b2
            uv_b[pl.ds(off_u + 128, 16)] = t1 * b0 + t3 * b1 + t5 * b2

        base = base0 + ci * CHUNK
        pending_out[ci] = pltpu.async_copy(
            uv_v[bi], out_hbm.at[pl.ds(2 * base, 2 * CHUNK)], sem_out[bi]
        )
    for d in pending_out.values():
        d.wait()


def kernel(points_bary, face_ids, faces_uvs_index):
    table = jnp.pad(
        faces_uvs_index.transpose(1, 2, 0).reshape(6, N_FACES),
        ((0, 0), (0, FPAD - N_FACES)),
    ).reshape(-1)
    fid = face_ids.astype(jnp.int32)
    bary_blocks = (
        points_bary.T.reshape(3, N_POINTS // 128, 128)
        .transpose(1, 0, 2)
        .reshape(-1)
    )
    out = _uv_kernel(table, fid, bary_blocks)
    return out.reshape(N_POINTS // 128, 2, 128).transpose(0, 2, 1).reshape(N_POINTS, 2)


# speedup vs baseline: 1.0202x; 1.0202x over previous
import functools

import jax
import jax.numpy as jnp
from jax import lax
from jax.experimental import pallas as pl
from jax.experimental.pallas import tpu as pltpu
from jax.experimental.pallas import tpu_sc as plsc

N_POINTS = 1048576
N_FACES = 1538

NUM_CORES = 2
NUM_SUBCORES = 16
NW = NUM_CORES * NUM_SUBCORES
PTS_PER_W = N_POINTS // NW
CHUNK = 4096
NBUF = 3
N_CHUNKS = PTS_PER_W // CHUNK
GROUPS = CHUNK // 16
FPAD = 1544

_mesh = plsc.VectorSubcoreMesh(
    core_axis_name="c", subcore_axis_name="s", num_cores=NUM_CORES
)


@functools.partial(
    pl.kernel,
    out_type=jax.ShapeDtypeStruct((2 * N_POINTS,), jnp.float32),
    mesh=_mesh,
    compiler_params=pltpu.CompilerParams(needs_layout_passes=False),
    scratch_types=[
        [pltpu.VMEM((FPAD,), jnp.float32) for _ in range(6)],
        [pltpu.VMEM((CHUNK,), jnp.int32) for _ in range(NBUF)],
        [pltpu.VMEM((3 * CHUNK,), jnp.float32) for _ in range(NBUF)],
        [pltpu.VMEM((2 * CHUNK,), jnp.float32) for _ in range(NBUF)],
        [pltpu.SemaphoreType.DMA for _ in range(NBUF)],
        [pltpu.SemaphoreType.DMA for _ in range(NBUF)],
        pltpu.SemaphoreType.DMA,
    ],
)
def _uv_kernel(
    table_hbm, fid_hbm, bary_hbm, out_hbm, table_v, fid_v, bary_v, uv_v,
    sem_in, sem_out, sem_tab,
):
    wid = lax.axis_index("s") * NUM_CORES + lax.axis_index("c")
    base0 = wid * PTS_PER_W
    t0_v, t1_v, t2_v, t3_v, t4_v, t5_v = table_v

    def start_in(ci, bi):
        base = base0 + ci * CHUNK
        return [
            pltpu.async_copy(fid_hbm.at[pl.ds(base, CHUNK)], fid_v[bi], sem_in[bi]),
            pltpu.async_copy(
                bary_hbm.at[pl.ds(3 * base, 3 * CHUNK)], bary_v[bi], sem_in[bi]
            ),
        ]

    pending_tab = [
        pltpu.async_copy(table_hbm.at[pl.ds(p * FPAD, FPAD)], table_v[p], sem_tab)
        for p in range(6)
    ]
    pending_in = {
        ci: start_in(ci, ci % NBUF) for ci in range(min(NBUF - 1, N_CHUNKS))
    }
    pending_out = {}
    for ci in range(N_CHUNKS):
        bi = ci % NBUF
        if ci + NBUF - 1 < N_CHUNKS:
            pending_in[ci + NBUF - 1] = start_in(ci + NBUF - 1, (ci + NBUF - 1) % NBUF)
        for d in pending_tab:
            d.wait()
        pending_tab = []
        for d in pending_in.pop(ci):
            d.wait()
        if ci - NBUF in pending_out:
            pending_out.pop(ci - NBUF).wait()

        bary_b, fid_b, uv_b = bary_v[bi], fid_v[bi], uv_v[bi]

        @plsc.parallel_loop(0, GROUPS, unroll=8)
        def group_body(g):
            fid = fid_b[pl.ds(g * 16, 16)]
            t0 = plsc.load_gather(t0_v, [fid])
            t1 = plsc.load_gather(t1_v, [fid])
            t2 = plsc.load_gather(t2_v, [fid])
            t3 = plsc.load_gather(t3_v, [fid])
            t4 = plsc.load_gather(t4_v, [fid])
            t5 = plsc.load_gather(t5_v, [fid])
            blk, r = g // 8, (g % 8) * 16
            off_b = blk * 384 + r
            b0 = bary_b[pl.ds(off_b, 16)]
            b1 = bary_b[pl.ds(off_b + 128, 16)]
            b2 = bary_b[pl.ds(off_b + 256, 16)]
            off_u = blk * 256 + r
            uv_b[pl.ds(off_u, 16)] = t0 * b0 + t2 * b1 + t4 * b2
            uv_b[pl.ds(off_u + 128, 16)] = t1 * b0 + t3 * b1 + t5 * b2

        base = base0 + ci * CHUNK
        pending_out[ci] = pltpu.async_copy(
            uv_v[bi], out_hbm.at[pl.ds(2 * base, 2 * CHUNK)], sem_out[bi]
        )
    for d in pending_out.values():
        d.wait()


def kernel(points_bary, face_ids, faces_uvs_index):
    table = jnp.pad(
        faces_uvs_index.transpose(1, 2, 0).reshape(6, N_FACES),
        ((0, 0), (0, FPAD - N_FACES)),
    ).reshape(-1)
    fid = face_ids.astype(jnp.int32)
    bary_blocks = (
        points_bary.T.reshape(3, N_POINTS // 128, 128)
        .transpose(1, 0, 2)
        .reshape(-1)
    )
    out = _uv_kernel(table, fid, bary_blocks)
    return out.reshape(N_POINTS // 128, 2, 128).transpose(0, 2, 1).reshape(N_POINTS, 2)


# speedup vs baseline: 1.0330x; 1.0125x over previous
import functools

import jax
import jax.numpy as jnp
from jax import lax
from jax.experimental import pallas as pl
from jax.experimental.pallas import tpu as pltpu
from jax.experimental.pallas import tpu_sc as plsc

N_POINTS = 1048576
N_FACES = 1538

NUM_CORES = 2
NUM_SUBCORES = 16
NW = NUM_CORES * NUM_SUBCORES
PTS_PER_W = N_POINTS // NW
CHUNK = 4096
NBUF = 2
N_CHUNKS = PTS_PER_W // CHUNK
GROUPS = CHUNK // 16
FPAD = 1544

_mesh = plsc.VectorSubcoreMesh(
    core_axis_name="c", subcore_axis_name="s", num_cores=NUM_CORES
)


@functools.partial(
    pl.kernel,
    out_type=jax.ShapeDtypeStruct((2 * N_POINTS,), jnp.float32),
    mesh=_mesh,
    compiler_params=pltpu.CompilerParams(needs_layout_passes=False),
    scratch_types=[
        [pltpu.VMEM((FPAD,), jnp.float32) for _ in range(6)],
        [pltpu.VMEM((CHUNK,), jnp.int32) for _ in range(NBUF)],
        [pltpu.VMEM((3 * CHUNK,), jnp.float32) for _ in range(NBUF)],
        [pltpu.VMEM((2 * CHUNK,), jnp.float32) for _ in range(NBUF)],
        [pltpu.SemaphoreType.DMA for _ in range(NBUF)],
        [pltpu.SemaphoreType.DMA for _ in range(NBUF)],
        pltpu.SemaphoreType.DMA,
    ],
)
def _uv_kernel(
    table_hbm, fid_hbm, bary_hbm, out_hbm, table_v, fid_v, bary_v, uv_v,
    sem_in, sem_out, sem_tab,
):
    wid = lax.axis_index("s") * NUM_CORES + lax.axis_index("c")
    base0 = wid * PTS_PER_W
    t0_v, t1_v, t2_v, t3_v, t4_v, t5_v = table_v

    def start_in(ci, bi):
        base = base0 + ci * CHUNK
        return [
            pltpu.async_copy(fid_hbm.at[pl.ds(base, CHUNK)], fid_v[bi], sem_in[bi]),
            pltpu.async_copy(
                bary_hbm.at[pl.ds(3 * base, 3 * CHUNK)], bary_v[bi], sem_in[bi]
            ),
        ]

    pending_tab = [
        pltpu.async_copy(table_hbm.at[pl.ds(p * FPAD, FPAD)], table_v[p], sem_tab)
        for p in range(6)
    ]
    pending_in = {
        ci: start_in(ci, ci % NBUF) for ci in range(min(NBUF - 1, N_CHUNKS))
    }
    pending_out = {}
    for ci in range(N_CHUNKS):
        bi = ci % NBUF
        if ci + NBUF - 1 < N_CHUNKS:
            pending_in[ci + NBUF - 1] = start_in(ci + NBUF - 1, (ci + NBUF - 1) % NBUF)
        for d in pending_tab:
            d.wait()
        pending_tab = []
        for d in pending_in.pop(ci):
            d.wait()
        if ci - NBUF in pending_out:
            pending_out.pop(ci - NBUF).wait()

        bary_b, fid_b, uv_b = bary_v[bi], fid_v[bi], uv_v[bi]

        @plsc.parallel_loop(0, GROUPS, unroll=8)
        def group_body(g):
            fid = fid_b[pl.ds(g * 16, 16)]
            t0 = plsc.load_gather(t0_v, [fid])
            t1 = plsc.load_gather(t1_v, [fid])
            t2 = plsc.load_gather(t2_v, [fid])
            t3 = plsc.load_gather(t3_v, [fid])
            t4 = plsc.load_gather(t4_v, [fid])
            t5 = plsc.load_gather(t5_v, [fid])
            blk, r = g // 8, (g % 8) * 16
            off_b = blk * 384 + r
            b0 = bary_b[pl.ds(off_b, 16)]
            b1 = bary_b[pl.ds(off_b + 128, 16)]
            b2 = bary_b[pl.ds(off_b + 256, 16)]
            off_u = blk * 256 + r
            uv_b[pl.ds(off_u, 16)] = t0 * b0 + t2 * b1 + t4 * b2
            uv_b[pl.ds(off_u + 128, 16)] = t1 * b0 + t3 * b1 + t5 * b2

        base = base0 + ci * CHUNK
        pending_out[ci] = pltpu.async_copy(
            uv_v[bi], out_hbm.at[pl.ds(2 * base, 2 * CHUNK)], sem_out[bi]
        )
    for d in pending_out.values():
        d.wait()


def kernel(points_bary, face_ids, faces_uvs_index):
    table = jnp.pad(
        faces_uvs_index.transpose(1, 2, 0).reshape(6, N_FACES),
        ((0, 0), (0, FPAD - N_FACES)),
    ).reshape(-1)
    fid = face_ids.astype(jnp.int32)
    bary_blocks = (
        points_bary.T.reshape(3, N_POINTS // 128, 128)
        .transpose(1, 0, 2)
        .reshape(-1)
    )
    out = _uv_kernel(table, fid, bary_blocks)
    return out.reshape(N_POINTS // 128, 2, 128).transpose(0, 2, 1).reshape(N_POINTS, 2)
